# Initial kernel scaffold; baseline (speedup 1.0000x reference)
import jax
import jax.numpy as jnp
from jax.experimental import pallas as pl


def kernel(query_length, key_length, rel_bias_table):
    raise NotImplementedError("write your pallas kernel here")


# speedup vs baseline: 159.9521x; 159.9521x over previous
import math
import functools

import jax
import jax.numpy as jnp
from jax import lax
from jax.experimental import pallas as pl
from jax.experimental.pallas import tpu as pltpu

NUM_BUCKETS = 32
MAX_DISTANCE = 128
NUM_HEADS = 16
SEQ = 4096

R = 128
LROW = 8064
EXT = 8192


def _bias_vec(table_col):
    t = lax.broadcasted_iota(jnp.int32, (1, EXT), 1)
    d = t - (SEQ - 1)
    n = -d
    ret = jnp.where(n < 0, NUM_BUCKETS // 2, 0)
    na = jnp.abs(n)
    max_exact = NUM_BUCKETS // 4
    is_small = na < max_exact
    naf = na.astype(jnp.float32)
    val = max_exact + (
        jnp.log(naf / max_exact)
        / math.log(MAX_DISTANCE / max_exact)
        * (NUM_BUCKETS // 2 - max_exact)
    ).astype(jnp.int32)
    val = jnp.minimum(val, NUM_BUCKETS // 2 - 1)
    bucket = ret + jnp.where(is_small, na, val)
    acc = jnp.zeros((1, EXT), jnp.float32)
    for b in range(NUM_BUCKETS):
        acc = jnp.where(bucket == b, table_col[b], acc)
    return acc


def _tc_body(table_ref, out_ref, bias_ref):
    h = pl.program_id(0)
    g = pl.program_id(1)

    @pl.when(g == 0)
    def _():
        vec = _bias_vec(table_ref[0, 0, :])
        for p in range(R):
            bias_ref[p, :] = vec[0, (R - 1 - p):(R - 1 - p) + LROW]

    i0 = g * R
    u0 = (SEQ - R) - i0
    out_ref[0, :, :] = bias_ref[:, pl.ds(u0, SEQ)]


def _tc_call(table3, interpret=False):
    return pl.pallas_call(
        _tc_body,
        grid=(NUM_HEADS, SEQ // R),
        in_specs=[pl.BlockSpec((1, 1, NUM_BUCKETS), lambda h, g: (h, 0, 0))],
        out_specs=pl.BlockSpec((1, R, SEQ), lambda h, g: (h, g, 0)),
        out_shape=jax.ShapeDtypeStruct((NUM_HEADS, SEQ, SEQ), jnp.float32),
        scratch_shapes=[pltpu.VMEM((R, LROW), jnp.float32)],
        interpret=interpret,
    )(table3)


def kernel(query_length, key_length, rel_bias_table):
    del query_length, key_length
    table3 = rel_bias_table.T.reshape(NUM_HEADS, 1, NUM_BUCKETS)
    out = _tc_call(table3)
    return out[None]
